# baseline (speedup 1.0000x reference)
import functools

import jax
import jax.numpy as jnp
from jax import lax
from jax.experimental import pallas as pl
from jax.experimental.pallas import tpu as pltpu
from jax.experimental.pallas import tpu_sc as plsc

N = 10000
E = 320000
D = 128
EPS = 1e-5

NC = 2
NS = 16
NW = NC * NS
L = 16

NPT = 320
NT = NPT * NW
BLK = 4000
NB = E // BLK
G = 128
CSZ = BLK + G + 2 * L
DUMMY = 1 << 29
FGRP = D // L

_mesh = plsc.VectorSubcoreMesh(
    core_axis_name="c", subcore_axis_name="s", num_cores=NC, num_subcores=NS
)


@functools.partial(
    pl.kernel,
    compiler_params=pltpu.CompilerParams(needs_layout_passes=False),
    out_type=[
        jax.ShapeDtypeStruct((NT, D), jnp.float32),
        jax.ShapeDtypeStruct((NT * L,), jnp.float32),
        jax.ShapeDtypeStruct((NT, D), jnp.float32),
    ],
    mesh=_mesh,
    scratch_types=[
        pltpu.VMEM((BLK,), jnp.int32),
        pltpu.VMEM((BLK,), jnp.int32),
        pltpu.VMEM((CSZ,), jnp.int32),
        pltpu.VMEM((CSZ,), jnp.int32),
        pltpu.VMEM((G, D), jnp.float32),
        pltpu.VMEM((NPT + 1, D), jnp.float32),
        pltpu.VMEM((NPT + 1, D), jnp.float32),
        pltpu.VMEM(((NPT + 1) * L,), jnp.float32),
        pltpu.SemaphoreType.DMA,
    ],
)
def _sc_aggregate(src_h, dst_h, h_h,
                  sum_o, cnt_o, max_o,
                  srcb, dstb, cs, cd, rows, sumacc, maxacc, cntacc, sem):
    c = lax.axis_index("c")
    s = lax.axis_index("s")
    wid = c * NS + s
    lo = wid * NPT

    neg = jnp.full((L,), -jnp.inf, jnp.float32)
    zrow = jnp.zeros((L,), jnp.float32)

    def init_acc(i, _):
        for f in range(FGRP):
            sumacc[i, pl.ds(f * L, L)] = zrow
            maxacc[i, pl.ds(f * L, L)] = neg
        cntacc[pl.ds(i * L, L)] = zrow
        return 0

    lax.fori_loop(0, NPT + 1, init_acc, 0)

    iota = lax.broadcasted_iota(jnp.int32, (L,), 0)
    zero16 = jnp.zeros((L,), jnp.int32)
    dummy16 = jnp.full((L,), DUMMY, jnp.int32)
    one16 = jnp.full((L,), 1.0, jnp.float32)

    def block_body(bi, _):
        e0 = bi * BLK
        pltpu.sync_copy(src_h.at[pl.ds(e0, BLK)], srcb)
        pltpu.sync_copy(dst_h.at[pl.ds(e0, BLK)], dstb)

        def comp(i, cnt):
            d = dstb[pl.ds(i * L, L)]
            sv = srcb[pl.ds(i * L, L)]
            m = (d >= lo) & (d < lo + NPT)
            csum = plsc.cumsum(m.astype(jnp.int32))
            pos = jnp.maximum(cnt + csum - 1, 0)
            plsc.store_scatter(cs, [pos], sv, mask=m)
            plsc.store_scatter(cd, [pos], d, mask=m)
            return cnt + csum[L - 1]

        cnt = lax.fori_loop(0, BLK // L, comp, 0)

        base = (cnt // L) * L
        for k in range(G // L + 1):
            lanes = base + k * L + iota
            m = lanes >= cnt
            plsc.store_scatter(cs, [lanes], zero16, mask=m)
            plsc.store_scatter(cd, [lanes], dummy16, mask=m)

        def chunk(g, _):
            g0 = g * G
            descs = [
                pltpu.async_copy(
                    h_h.at[cs.at[pl.ds(g0 + j * L, L)]],
                    rows.at[pl.ds(j * L, L)],
                    sem,
                )
                for j in range(G // L)
            ]
            for dsc in descs:
                dsc.wait()

            def grp(t, _):
                t0 = t * L
                dv = cd[pl.ds(g0 + t0, L)] - lo
                dvc = jnp.clip(dv, 0, NPT)
                for l in range(L):
                    dj = dvc[l]
                    cntacc[pl.ds(dj * L, L)] += one16
                    for f in range(FGRP):
                        r = rows[t0 + l, pl.ds(f * L, L)]
                        sumacc[dj, pl.ds(f * L, L)] += r
                        a = maxacc[dj, pl.ds(f * L, L)]
                        maxacc[dj, pl.ds(f * L, L)] = jnp.maximum(a, r)
                return 0

            lax.fori_loop(0, G // L, grp, 0)
            return 0

        ng = (cnt + G - 1) // G
        lax.fori_loop(0, ng, chunk, 0)
        return 0

    lax.fori_loop(0, NB, block_body, 0)

    pltpu.sync_copy(sumacc.at[pl.ds(0, NPT)], sum_o.at[pl.ds(lo, NPT)])
    pltpu.sync_copy(cntacc.at[pl.ds(0, NPT * L)], cnt_o.at[pl.ds(lo * L, NPT * L)])
    pltpu.sync_copy(maxacc.at[pl.ds(0, NPT)], max_o.at[pl.ds(lo, NPT)])


_ROWS_BLK = 1000
_GRID = N // _ROWS_BLK


def _tc_linear_body(p, cdeg, m, w1, w2, b2, sn, h2_ref, s1, s2):
    deg = cdeg[:, :1]
    mean = p[...] / jnp.maximum(deg, 1.0)
    mx = jnp.where(deg > 0.0, m[...], 0.0)
    h2 = (
        jnp.dot(mean, w1[...], preferred_element_type=jnp.float32)
        + jnp.dot(mx, w2[...], preferred_element_type=jnp.float32)
        + b2[...]
    ) * sn[...]
    h2_ref[...] = h2

    @pl.when(pl.program_id(0) == 0)
    def _():
        s1[...] = jnp.zeros_like(s1)
        s2[...] = jnp.zeros_like(s2)

    s1[...] += jnp.sum(h2, axis=0, keepdims=True)
    s2[...] += jnp.sum(h2 * h2, axis=0, keepdims=True)


def _tc_norm_body(h2, h, s1, s2, gamma2, beta2, out):
    mu = s1[...] / N
    var = s2[...] / N - mu * mu
    scale = gamma2[...] * lax.rsqrt(var + EPS)
    out[...] = h[...] + jnp.maximum(scale * (h2[...] - mu) + beta2[...], 0.0)


def kernel(h, e, eig, snorm_n, edge_index, W, b, gamma, beta):
    src = edge_index[0].astype(jnp.int32)
    dst = edge_index[1].astype(jnp.int32)

    sum_p, cnt_p, max_p = _sc_aggregate(src, dst, h)

    p = sum_p[:N]
    cdeg = cnt_p.reshape(NT, L)[:N]
    m = max_p[:N]

    rb = lambda i: (i, 0)
    fb = lambda i: (0, 0)
    h2, s1, s2 = pl.pallas_call(
        _tc_linear_body,
        grid=(_GRID,),
        in_specs=[
            pl.BlockSpec((_ROWS_BLK, D), rb),
            pl.BlockSpec((_ROWS_BLK, L), rb),
            pl.BlockSpec((_ROWS_BLK, D), rb),
            pl.BlockSpec((D, D), fb),
            pl.BlockSpec((D, D), fb),
            pl.BlockSpec((1, D), fb),
            pl.BlockSpec((_ROWS_BLK, 1), rb),
        ],
        out_specs=[
            pl.BlockSpec((_ROWS_BLK, D), rb),
            pl.BlockSpec((1, D), fb),
            pl.BlockSpec((1, D), fb),
        ],
        out_shape=[
            jax.ShapeDtypeStruct((N, D), jnp.float32),
            jax.ShapeDtypeStruct((1, D), jnp.float32),
            jax.ShapeDtypeStruct((1, D), jnp.float32),
        ],
    )(p, cdeg, m, W[:D], W[D:], b.reshape(1, D), snorm_n)

    out = pl.pallas_call(
        _tc_norm_body,
        grid=(_GRID,),
        in_specs=[
            pl.BlockSpec((_ROWS_BLK, D), rb),
            pl.BlockSpec((_ROWS_BLK, D), rb),
            pl.BlockSpec((1, D), fb),
            pl.BlockSpec((1, D), fb),
            pl.BlockSpec((1, D), fb),
            pl.BlockSpec((1, D), fb),
        ],
        out_specs=pl.BlockSpec((_ROWS_BLK, D), rb),
        out_shape=jax.ShapeDtypeStruct((N, D), jnp.float32),
    )(h2, h, s1, s2, gamma.reshape(1, D), beta.reshape(1, D))

    return out


# speedup vs baseline: 3.3827x; 3.3827x over previous
import functools

import jax
import jax.numpy as jnp
from jax import lax
from jax.experimental import pallas as pl
from jax.experimental.pallas import tpu as pltpu
from jax.experimental.pallas import tpu_sc as plsc

N = 10000
E = 320000
D = 128
HD = D // 2
EPS = 1e-5

NC = 2
NS = 16
NW = NC * NS
L = 16

NPT = 320
NT = NPT * NW
BLK = 4000
NB = E // BLK
G = 128
CSZ = BLK + G + 2 * L
DUMMY = 1 << 29
FG = HD // L

_mesh = plsc.VectorSubcoreMesh(
    core_axis_name="c", subcore_axis_name="s", num_cores=NC, num_subcores=NS
)


@functools.partial(
    pl.kernel,
    compiler_params=pltpu.CompilerParams(needs_layout_passes=False),
    out_type=[
        jax.ShapeDtypeStruct((NT * HD,), jnp.float32),
        jax.ShapeDtypeStruct((NT * L,), jnp.float32),
        jax.ShapeDtypeStruct((NT * HD,), jnp.float32),
    ],
    mesh=_mesh,
    scratch_types=[
        pltpu.VMEM((BLK,), jnp.int32),
        pltpu.VMEM((BLK,), jnp.int32),
        pltpu.VMEM((CSZ,), jnp.int32),
        pltpu.VMEM((CSZ,), jnp.int32),
        pltpu.VMEM((G, D), jnp.float32),
        pltpu.VMEM(((NPT + 1) * HD,), jnp.float32),
        pltpu.VMEM(((NPT + 1) * HD,), jnp.float32),
        pltpu.VMEM(((NPT + 1) * L,), jnp.float32),
        pltpu.VMEM_SHARED((N // 2, D), jnp.float32),
        pltpu.SemaphoreType.DMA,
    ],
)
def _sc_half(src_h, dst_h, h_h,
             sum_o, cnt_o, max_o,
             srcb, dstb, cs, cd, rows, sumacc, maxacc, cntacc, sp_h, sem):
    c = lax.axis_index("c")
    s = lax.axis_index("s")
    wid = c * NS + s
    lo = wid * NPT

    neg = jnp.full((L,), -jnp.inf, jnp.float32)
    zrow = jnp.zeros((L,), jnp.float32)

    @pl.when(s == 0)
    def _():
        pltpu.sync_copy(h_h, sp_h)

    def init_acc(i, _):
        for f in range(FG):
            sumacc[pl.ds(i * HD + f * L, L)] = zrow
            maxacc[pl.ds(i * HD + f * L, L)] = neg
        cntacc[pl.ds(i * L, L)] = zrow
        return 0

    lax.fori_loop(0, NPT + 1, init_acc, 0)
    plsc.subcore_barrier()

    iota = lax.broadcasted_iota(jnp.int32, (L,), 0)
    padsrc16 = wid * 128 + iota * 4
    dummy16 = jnp.full((L,), 2 * NPT, jnp.int32)
    one16 = jnp.full((L,), 1.0, jnp.float32)

    def block_body(bi, _):
        e0 = bi * BLK
        pltpu.sync_copy(src_h.at[pl.ds(e0, BLK)], srcb)
        pltpu.sync_copy(dst_h.at[pl.ds(e0, BLK)], dstb)

        def comp(i, cnt):
            d = dstb[pl.ds(i * L, L)]
            sv = srcb[pl.ds(i * L, L)]
            m = (d >= lo) & (d < lo + NPT)
            csum = plsc.cumsum(m.astype(jnp.int32))
            pos = jnp.maximum(cnt + csum - 1, 0)
            plsc.store_scatter(cs, [pos], sv // 2, mask=m)
            plsc.store_scatter(cd, [pos], (d - lo) * 2 + (sv & 1), mask=m)
            return cnt + csum[L - 1]

        cnt = lax.fori_loop(0, BLK // L, comp, 0)

        base = (cnt // L) * L
        for k in range(G // L + 1):
            lanes = base + k * L + iota
            m = lanes >= cnt
            plsc.store_scatter(cs, [lanes], padsrc16, mask=m)
            plsc.store_scatter(cd, [lanes], dummy16, mask=m)

        def chunk(g, _):
            g0 = g * G
            descs = [
                pltpu.async_copy(
                    sp_h.at[cs.at[pl.ds(g0 + j * L, L)]],
                    rows.at[pl.ds(j * L, L)],
                    sem,
                )
                for j in range(G // L)
            ]
            for dsc in descs:
                dsc.wait()

            def grp(t, _):
                t0 = t * L
                dvp = cd[pl.ds(g0 + t0, L)]
                for l in range(L):
                    v = dvp[l]
                    dj = v // 2
                    poff = (v & 1) * HD
                    cntacc[pl.ds(dj * L, L)] += one16
                    for f in range(FG):
                        r = rows[t0 + l, pl.ds(poff + f * L, L)]
                        sumacc[pl.ds(dj * HD + f * L, L)] += r
                        a = maxacc[pl.ds(dj * HD + f * L, L)]
                        maxacc[pl.ds(dj * HD + f * L, L)] = jnp.maximum(a, r)
                return 0

            lax.fori_loop(0, G // L, grp, 0)
            return 0

        ng = (cnt + G - 1) // G
        lax.fori_loop(0, ng, chunk, 0)
        return 0

    lax.fori_loop(0, NB, block_body, 0)

    pltpu.sync_copy(sumacc.at[pl.ds(0, NPT * HD)], sum_o.at[pl.ds(lo * HD, NPT * HD)])
    pltpu.sync_copy(cntacc.at[pl.ds(0, NPT * L)], cnt_o.at[pl.ds(lo * L, NPT * L)])
    pltpu.sync_copy(maxacc.at[pl.ds(0, NPT * HD)], max_o.at[pl.ds(lo * HD, NPT * HD)])


_ROWS_BLK = 1000
_GRID = N // _ROWS_BLK


def _tc_linear_body(s0, s1, m0, m1, cdeg, w00, w01, w10, w11, b2, sn,
                    h2_ref, o1, o2):
    deg = cdeg[:, :1]
    inv = 1.0 / jnp.maximum(deg, 1.0)
    pos = deg > 0.0
    mean0 = s0[...] * inv
    mean1 = s1[...] * inv
    mx0 = jnp.where(pos, m0[...], 0.0)
    mx1 = jnp.where(pos, m1[...], 0.0)
    h2 = (
        jnp.dot(mean0, w00[...], preferred_element_type=jnp.float32)
        + jnp.dot(mean1, w01[...], preferred_element_type=jnp.float32)
        + jnp.dot(mx0, w10[...], preferred_element_type=jnp.float32)
        + jnp.dot(mx1, w11[...], preferred_element_type=jnp.float32)
        + b2[...]
    ) * sn[...]
    h2_ref[...] = h2

    @pl.when(pl.program_id(0) == 0)
    def _():
        o1[...] = jnp.zeros_like(o1)
        o2[...] = jnp.zeros_like(o2)

    o1[...] += jnp.sum(h2, axis=0, keepdims=True)
    o2[...] += jnp.sum(h2 * h2, axis=0, keepdims=True)


def _tc_norm_body(h2, h, s1, s2, gamma2, beta2, out):
    mu = s1[...] / N
    var = s2[...] / N - mu * mu
    scale = gamma2[...] * lax.rsqrt(var + EPS)
    out[...] = h[...] + jnp.maximum(scale * (h2[...] - mu) + beta2[...], 0.0)


def kernel(h, e, eig, snorm_n, edge_index, W, b, gamma, beta):
    src = edge_index[0].astype(jnp.int32)
    dst = edge_index[1].astype(jnp.int32)

    s0f, c0f, m0f = _sc_half(src, dst, h[:, :HD].reshape(N // 2, D))
    hp1, _ = lax.optimization_barrier((h[:, HD:].reshape(N // 2, D), c0f))
    s1f, _, m1f = _sc_half(src, dst, hp1)

    s0 = s0f.reshape(NT, HD)[:N]
    s1 = s1f.reshape(NT, HD)[:N]
    m0 = m0f.reshape(NT, HD)[:N]
    m1 = m1f.reshape(NT, HD)[:N]
    cdeg = c0f.reshape(NT, L)[:N]

    rb = lambda i: (i, 0)
    fb = lambda i: (0, 0)
    h2, t1, t2 = pl.pallas_call(
        _tc_linear_body,
        grid=(_GRID,),
        in_specs=[
            pl.BlockSpec((_ROWS_BLK, HD), rb),
            pl.BlockSpec((_ROWS_BLK, HD), rb),
            pl.BlockSpec((_ROWS_BLK, HD), rb),
            pl.BlockSpec((_ROWS_BLK, HD), rb),
            pl.BlockSpec((_ROWS_BLK, L), rb),
            pl.BlockSpec((HD, D), fb),
            pl.BlockSpec((HD, D), fb),
            pl.BlockSpec((HD, D), fb),
            pl.BlockSpec((HD, D), fb),
            pl.BlockSpec((1, D), fb),
            pl.BlockSpec((_ROWS_BLK, 1), rb),
        ],
        out_specs=[
            pl.BlockSpec((_ROWS_BLK, D), rb),
            pl.BlockSpec((1, D), fb),
            pl.BlockSpec((1, D), fb),
        ],
        out_shape=[
            jax.ShapeDtypeStruct((N, D), jnp.float32),
            jax.ShapeDtypeStruct((1, D), jnp.float32),
            jax.ShapeDtypeStruct((1, D), jnp.float32),
        ],
    )(s0, s1, m0, m1, cdeg, W[:HD], W[HD:D], W[D:D + HD], W[D + HD:],
      b.reshape(1, D), snorm_n)

    out = pl.pallas_call(
        _tc_norm_body,
        grid=(_GRID,),
        in_specs=[
            pl.BlockSpec((_ROWS_BLK, D), rb),
            pl.BlockSpec((_ROWS_BLK, D), rb),
            pl.BlockSpec((1, D), fb),
            pl.BlockSpec((1, D), fb),
            pl.BlockSpec((1, D), fb),
            pl.BlockSpec((1, D), fb),
        ],
        out_specs=pl.BlockSpec((_ROWS_BLK, D), rb),
        out_shape=jax.ShapeDtypeStruct((N, D), jnp.float32),
    )(h2, h, t1, t2, gamma.reshape(1, D), beta.reshape(1, D))

    return out


# speedup vs baseline: 4.1395x; 1.2237x over previous
import functools

import jax
import jax.numpy as jnp
from jax import lax
from jax.experimental import pallas as pl
from jax.experimental.pallas import tpu as pltpu
from jax.experimental.pallas import tpu_sc as plsc

N = 10000
E = 320000
D = 128
HD = D // 2
EPS = 1e-5

NC = 2
NS = 16
NW = NC * NS
L = 16

NPT = 320
NT = NPT * NW
BLK = 4000
NB = E // BLK
G = 64
CSZ = BLK + G + 2 * L
DUMMY = 1 << 29
FG = HD // L

_mesh = plsc.VectorSubcoreMesh(
    core_axis_name="c", subcore_axis_name="s", num_cores=NC, num_subcores=NS
)


@functools.partial(
    pl.kernel,
    compiler_params=pltpu.CompilerParams(needs_layout_passes=False),
    out_type=[
        jax.ShapeDtypeStruct((NT * HD,), jnp.float32),
        jax.ShapeDtypeStruct((NT * L,), jnp.float32),
        jax.ShapeDtypeStruct((NT * HD,), jnp.float32),
    ],
    mesh=_mesh,
    scratch_types=[
        pltpu.VMEM((BLK,), jnp.int32),
        pltpu.VMEM((BLK,), jnp.int32),
        pltpu.VMEM((CSZ,), jnp.int32),
        pltpu.VMEM((CSZ,), jnp.int32),
        pltpu.VMEM((2 * G, D), jnp.float32),
        pltpu.VMEM(((NPT + 1) * HD,), jnp.float32),
        pltpu.VMEM(((NPT + 1) * HD,), jnp.float32),
        pltpu.VMEM(((NPT + 1) * L,), jnp.float32),
        pltpu.VMEM_SHARED((N // 2, D), jnp.float32),
        pltpu.SemaphoreType.DMA,
        pltpu.SemaphoreType.DMA,
    ],
)
def _sc_half(src_h, dst_h, h_h,
             sum_o, cnt_o, max_o,
             srcb, dstb, cs, cd, rows, sumacc, maxacc, cntacc, sp_h, sem, sem2):
    c = lax.axis_index("c")
    s = lax.axis_index("s")
    wid = c * NS + s
    lo = wid * NPT

    neg = jnp.full((L,), -jnp.inf, jnp.float32)
    zrow = jnp.zeros((L,), jnp.float32)

    @pl.when(s == 0)
    def _():
        pltpu.sync_copy(h_h, sp_h)

    def init_acc(i, _):
        for f in range(FG):
            sumacc[pl.ds(i * HD + f * L, L)] = zrow
            maxacc[pl.ds(i * HD + f * L, L)] = neg
        cntacc[pl.ds(i * L, L)] = zrow
        return 0

    lax.fori_loop(0, NPT + 1, init_acc, 0)
    plsc.subcore_barrier()

    iota = lax.broadcasted_iota(jnp.int32, (L,), 0)
    padsrc16 = wid * 128 + iota * 4
    dummy16 = jnp.full((L,), 2 * NPT, jnp.int32)
    one16 = jnp.full((L,), 1.0, jnp.float32)

    def block_body(bi, _):
        e0 = bi * BLK
        d1 = pltpu.async_copy(src_h.at[pl.ds(e0, BLK)], srcb, sem2)
        d2 = pltpu.async_copy(dst_h.at[pl.ds(e0, BLK)], dstb, sem2)
        d1.wait()
        d2.wait()

        def comp(i, cnt):
            d = dstb[pl.ds(i * L, L)]
            sv = srcb[pl.ds(i * L, L)]
            m = (d >= lo) & (d < lo + NPT)
            csum = plsc.cumsum(m.astype(jnp.int32))
            pos = jnp.maximum(cnt + csum - 1, 0)
            plsc.store_scatter(cs, [pos], sv // 2, mask=m)
            plsc.store_scatter(cd, [pos], (d - lo) * 2 + (sv & 1), mask=m)
            return cnt + csum[L - 1]

        cnt = lax.fori_loop(0, BLK // L, comp, 0, unroll=2)

        base = (cnt // L) * L
        for k in range(G // L + 1):
            lanes = base + k * L + iota
            m = lanes >= cnt
            plsc.store_scatter(cs, [lanes], padsrc16, mask=m)
            plsc.store_scatter(cd, [lanes], dummy16, mask=m)

        ng = (cnt + G - 1) // G

        def fire(g):
            gb = (g % 2) * G
            for j in range(G // L):
                pltpu.async_copy(
                    sp_h.at[cs.at[pl.ds(g * G + j * L, L)]],
                    rows.at[pl.ds(gb + j * L, L)],
                    sem,
                )

        def drain(g):
            gb = (g % 2) * G
            for j in range(G // L):
                pltpu.make_async_copy(
                    sp_h.at[cs.at[pl.ds(g * G + j * L, L)]],
                    rows.at[pl.ds(gb + j * L, L)],
                    sem,
                ).wait()

        @pl.when(ng > 0)
        def _():
            fire(0)

        def chunk(g, _):
            drain(g)

            @pl.when(g + 1 < ng)
            def _():
                fire(g + 1)

            gb = (g % 2) * G

            def grp(t, _):
                t0 = t * L
                dvp = cd[pl.ds(g * G + t0, L)]
                for l in range(L):
                    v = dvp[l]
                    dj = v // 2
                    poff = (v & 1) * HD
                    cntacc[pl.ds(dj * L, L)] += one16
                    for f in range(FG):
                        r = rows[gb + t0 + l, pl.ds(poff + f * L, L)]
                        sumacc[pl.ds(dj * HD + f * L, L)] += r
                        a = maxacc[pl.ds(dj * HD + f * L, L)]
                        maxacc[pl.ds(dj * HD + f * L, L)] = jnp.maximum(a, r)
                return 0

            lax.fori_loop(0, G // L, grp, 0)
            return 0

        lax.fori_loop(0, ng, chunk, 0)
        return 0

    lax.fori_loop(0, NB, block_body, 0)

    pltpu.sync_copy(sumacc.at[pl.ds(0, NPT * HD)], sum_o.at[pl.ds(lo * HD, NPT * HD)])
    pltpu.sync_copy(cntacc.at[pl.ds(0, NPT * L)], cnt_o.at[pl.ds(lo * L, NPT * L)])
    pltpu.sync_copy(maxacc.at[pl.ds(0, NPT * HD)], max_o.at[pl.ds(lo * HD, NPT * HD)])


_ROWS_BLK = 1000
_GRID = N // _ROWS_BLK


def _tc_linear_body(s0, s1, m0, m1, cdeg, w00, w01, w10, w11, b2, sn,
                    h2_ref, o1, o2):
    deg = cdeg[:, :1]
    inv = 1.0 / jnp.maximum(deg, 1.0)
    pos = deg > 0.0
    mean0 = s0[...] * inv
    mean1 = s1[...] * inv
    mx0 = jnp.where(pos, m0[...], 0.0)
    mx1 = jnp.where(pos, m1[...], 0.0)
    h2 = (
        jnp.dot(mean0, w00[...], preferred_element_type=jnp.float32)
        + jnp.dot(mean1, w01[...], preferred_element_type=jnp.float32)
        + jnp.dot(mx0, w10[...], preferred_element_type=jnp.float32)
        + jnp.dot(mx1, w11[...], preferred_element_type=jnp.float32)
        + b2[...]
    ) * sn[...]
    h2_ref[...] = h2

    @pl.when(pl.program_id(0) == 0)
    def _():
        o1[...] = jnp.zeros_like(o1)
        o2[...] = jnp.zeros_like(o2)

    o1[...] += jnp.sum(h2, axis=0, keepdims=True)
    o2[...] += jnp.sum(h2 * h2, axis=0, keepdims=True)


def _tc_norm_body(h2, h, s1, s2, gamma2, beta2, out):
    mu = s1[...] / N
    var = s2[...] / N - mu * mu
    scale = gamma2[...] * lax.rsqrt(var + EPS)
    out[...] = h[...] + jnp.maximum(scale * (h2[...] - mu) + beta2[...], 0.0)


def kernel(h, e, eig, snorm_n, edge_index, W, b, gamma, beta):
    src = edge_index[0].astype(jnp.int32)
    dst = edge_index[1].astype(jnp.int32)

    s0f, c0f, m0f = _sc_half(src, dst, h[:, :HD].reshape(N // 2, D))
    hp1, _ = lax.optimization_barrier((h[:, HD:].reshape(N // 2, D), c0f))
    s1f, _, m1f = _sc_half(src, dst, hp1)

    s0 = s0f.reshape(NT, HD)[:N]
    s1 = s1f.reshape(NT, HD)[:N]
    m0 = m0f.reshape(NT, HD)[:N]
    m1 = m1f.reshape(NT, HD)[:N]
    cdeg = c0f.reshape(NT, L)[:N]

    rb = lambda i: (i, 0)
    fb = lambda i: (0, 0)
    h2, t1, t2 = pl.pallas_call(
        _tc_linear_body,
        grid=(_GRID,),
        in_specs=[
            pl.BlockSpec((_ROWS_BLK, HD), rb),
            pl.BlockSpec((_ROWS_BLK, HD), rb),
            pl.BlockSpec((_ROWS_BLK, HD), rb),
            pl.BlockSpec((_ROWS_BLK, HD), rb),
            pl.BlockSpec((_ROWS_BLK, L), rb),
            pl.BlockSpec((HD, D), fb),
            pl.BlockSpec((HD, D), fb),
            pl.BlockSpec((HD, D), fb),
            pl.BlockSpec((HD, D), fb),
            pl.BlockSpec((1, D), fb),
            pl.BlockSpec((_ROWS_BLK, 1), rb),
        ],
        out_specs=[
            pl.BlockSpec((_ROWS_BLK, D), rb),
            pl.BlockSpec((1, D), fb),
            pl.BlockSpec((1, D), fb),
        ],
        out_shape=[
            jax.ShapeDtypeStruct((N, D), jnp.float32),
            jax.ShapeDtypeStruct((1, D), jnp.float32),
            jax.ShapeDtypeStruct((1, D), jnp.float32),
        ],
    )(s0, s1, m0, m1, cdeg, W[:HD], W[HD:D], W[D:D + HD], W[D + HD:],
      b.reshape(1, D), snorm_n)

    out = pl.pallas_call(
        _tc_norm_body,
        grid=(_GRID,),
        in_specs=[
            pl.BlockSpec((_ROWS_BLK, D), rb),
            pl.BlockSpec((_ROWS_BLK, D), rb),
            pl.BlockSpec((1, D), fb),
            pl.BlockSpec((1, D), fb),
            pl.BlockSpec((1, D), fb),
            pl.BlockSpec((1, D), fb),
        ],
        out_specs=pl.BlockSpec((_ROWS_BLK, D), rb),
        out_shape=jax.ShapeDtypeStruct((N, D), jnp.float32),
    )(h2, h, t1, t2, gamma.reshape(1, D), beta.reshape(1, D))

    return out


# speedup vs baseline: 4.8173x; 1.1637x over previous
import functools

import jax
import jax.numpy as jnp
from jax import lax
from jax.experimental import pallas as pl
from jax.experimental.pallas import tpu as pltpu
from jax.experimental.pallas import tpu_sc as plsc

N = 10000
E = 320000
D = 128
HD = D // 2
EPS = 1e-5

NC = 2
NS = 16
NW = NC * NS
L = 16

NPT = 320
NT = NPT * NW
BLK = 4000
NB = E // BLK
G = 64
CSZ = BLK + 2 * G + 2 * L
G2 = 2 * G
DUMMY = 1 << 29
FG = HD // L
EP = E + NB * 2 * G

_mesh = plsc.VectorSubcoreMesh(
    core_axis_name="c", subcore_axis_name="s", num_cores=NC, num_subcores=NS
)


@functools.partial(
    pl.kernel,
    compiler_params=pltpu.CompilerParams(needs_layout_passes=False),
    out_type=[
        jax.ShapeDtypeStruct((NT * HD,), jnp.float32),
        jax.ShapeDtypeStruct((NT * L,), jnp.float32),
        jax.ShapeDtypeStruct((NT * HD,), jnp.float32),
        jax.ShapeDtypeStruct((NW, EP), jnp.int32),
        jax.ShapeDtypeStruct((NW, EP), jnp.int32),
        jax.ShapeDtypeStruct((NW * G2,), jnp.int32),
    ],
    mesh=_mesh,
    scratch_types=[
        pltpu.VMEM((BLK,), jnp.int32),
        pltpu.VMEM((BLK,), jnp.int32),
        pltpu.VMEM((CSZ,), jnp.int32),
        pltpu.VMEM((CSZ,), jnp.int32),
        pltpu.VMEM((2 * G, D), jnp.float32),
        pltpu.VMEM(((NPT + 1) * HD,), jnp.float32),
        pltpu.VMEM(((NPT + 1) * HD,), jnp.float32),
        pltpu.VMEM(((NPT + 1) * L,), jnp.float32),
        pltpu.VMEM_SHARED((N // 2, D), jnp.float32),
        pltpu.SemaphoreType.DMA,
        pltpu.SemaphoreType.DMA,
    ],
)
def _sc_half(src_h, dst_h, h_h,
             sum_o, cnt_o, max_o, csx_o, cdx_o, ctot_o,
             srcb, dstb, cs, cd, rows, sumacc, maxacc, cntacc, sp_h, sem, sem2):
    c = lax.axis_index("c")
    s = lax.axis_index("s")
    wid = c * NS + s
    lo = wid * NPT

    neg = jnp.full((L,), -jnp.inf, jnp.float32)
    zrow = jnp.zeros((L,), jnp.float32)

    @pl.when(s == 0)
    def _():
        pltpu.sync_copy(h_h, sp_h)

    def init_acc(i, _):
        for f in range(FG):
            sumacc[pl.ds(i * HD + f * L, L)] = zrow
            maxacc[pl.ds(i * HD + f * L, L)] = neg
        cntacc[pl.ds(i * L, L)] = zrow
        return 0

    lax.fori_loop(0, NPT + 1, init_acc, 0)
    plsc.subcore_barrier()

    iota = lax.broadcasted_iota(jnp.int32, (L,), 0)
    padsrc16 = wid * 128 + iota * 4
    dummy16 = jnp.full((L,), 2 * NPT, jnp.int32)
    one16 = jnp.full((L,), 1.0, jnp.float32)

    def block_body(bi, ofs):
        e0 = bi * BLK
        d1 = pltpu.async_copy(src_h.at[pl.ds(e0, BLK)], srcb, sem2)
        d2 = pltpu.async_copy(dst_h.at[pl.ds(e0, BLK)], dstb, sem2)
        d1.wait()
        d2.wait()

        def comp(i, cnt):
            d = dstb[pl.ds(i * L, L)]
            sv = srcb[pl.ds(i * L, L)]
            m = (d >= lo) & (d < lo + NPT)
            csum = plsc.cumsum(m.astype(jnp.int32))
            pos = jnp.maximum(cnt + csum - 1, 0)
            plsc.store_scatter(cs, [pos], sv // 2, mask=m)
            plsc.store_scatter(cd, [pos], (d - lo) * 2 + (sv & 1), mask=m)
            return cnt + csum[L - 1]

        cnt = lax.fori_loop(0, BLK // L, comp, 0, unroll=2)

        base = (cnt // L) * L
        for k in range(2 * G // L + 1):
            lanes = base + k * L + iota
            m = lanes >= cnt
            plsc.store_scatter(cs, [lanes], padsrc16, mask=m)
            plsc.store_scatter(cd, [lanes], dummy16, mask=m)

        cnt128 = ((cnt + G2 - 1) // G2) * G2
        ng = cnt128 // G

        def fire(g):
            gb = (g % 2) * G
            for j in range(G // L):
                pltpu.async_copy(
                    sp_h.at[cs.at[pl.ds(g * G + j * L, L)]],
                    rows.at[pl.ds(gb + j * L, L)],
                    sem,
                )

        def drain(g):
            gb = (g % 2) * G
            for j in range(G // L):
                pltpu.make_async_copy(
                    sp_h.at[cs.at[pl.ds(g * G + j * L, L)]],
                    rows.at[pl.ds(gb + j * L, L)],
                    sem,
                ).wait()

        @pl.when(ng > 0)
        def _():
            fire(0)

        def chunk(g, _):
            drain(g)

            @pl.when(g + 1 < ng)
            def _():
                fire(g + 1)

            gb = (g % 2) * G

            def grp(t, _):
                t0 = t * L
                dvp = cd[pl.ds(g * G + t0, L)]
                for l in range(L):
                    v = dvp[l]
                    dj = v // 2
                    poff = (v & 1) * HD
                    cntacc[pl.ds(dj * L, L)] += one16
                    for f in range(FG):
                        r = rows[gb + t0 + l, pl.ds(poff + f * L, L)]
                        sumacc[pl.ds(dj * HD + f * L, L)] += r
                        a = maxacc[pl.ds(dj * HD + f * L, L)]
                        maxacc[pl.ds(dj * HD + f * L, L)] = jnp.maximum(a, r)
                return 0

            lax.fori_loop(0, G // L, grp, 0)
            return 0

        lax.fori_loop(0, ng, chunk, 0)

        def wb(w, _):
            off = pl.multiple_of(ofs + w * G2, G2)
            pltpu.sync_copy(cs.at[pl.ds(w * G2, G2)],
                            csx_o.at[wid].at[pl.ds(off, G2)])
            pltpu.sync_copy(cd.at[pl.ds(w * G2, G2)],
                            cdx_o.at[wid].at[pl.ds(off, G2)])
            return 0

        lax.fori_loop(0, cnt128 // G2, wb, 0)
        return ofs + cnt128

    total = lax.fori_loop(0, NB, block_body, 0)
    tvec = jnp.full((L,), 1, jnp.int32) * total
    for k in range(G2 // L):
        srcb[pl.ds(k * L, L)] = tvec
    pltpu.sync_copy(srcb.at[pl.ds(0, G2)], ctot_o.at[pl.ds(wid * G2, G2)])

    pltpu.sync_copy(sumacc.at[pl.ds(0, NPT * HD)], sum_o.at[pl.ds(lo * HD, NPT * HD)])
    pltpu.sync_copy(cntacc.at[pl.ds(0, NPT * L)], cnt_o.at[pl.ds(lo * L, NPT * L)])
    pltpu.sync_copy(maxacc.at[pl.ds(0, NPT * HD)], max_o.at[pl.ds(lo * HD, NPT * HD)])


@functools.partial(
    pl.kernel,
    compiler_params=pltpu.CompilerParams(needs_layout_passes=False),
    out_type=[
        jax.ShapeDtypeStruct((NT * HD,), jnp.float32),
        jax.ShapeDtypeStruct((NT * HD,), jnp.float32),
    ],
    mesh=_mesh,
    scratch_types=[
        pltpu.VMEM((2 * G2,), jnp.int32),
        pltpu.VMEM((2 * G2,), jnp.int32),
        pltpu.VMEM((2 * G2, D), jnp.float32),
        pltpu.VMEM(((NPT + 1) * HD,), jnp.float32),
        pltpu.VMEM(((NPT + 1) * HD,), jnp.float32),
        pltpu.VMEM((G2,), jnp.int32),
        pltpu.VMEM_SHARED((N // 2, D), jnp.float32),
        pltpu.SemaphoreType.DMA,
        pltpu.SemaphoreType.DMA,
    ],
)
def _sc_pass2(csx_h, cdx_h, ctot_h, h_h,
              sum_o, max_o,
              csb, cdb, rows, sumacc, maxacc, tot_v, sp_h, sem, sem2):
    c = lax.axis_index("c")
    s = lax.axis_index("s")
    wid = c * NS + s
    lo = wid * NPT

    neg = jnp.full((L,), -jnp.inf, jnp.float32)
    zrow = jnp.zeros((L,), jnp.float32)

    @pl.when(s == 0)
    def _():
        pltpu.sync_copy(h_h, sp_h)

    def init_acc(i, _):
        for f in range(FG):
            sumacc[pl.ds(i * HD + f * L, L)] = zrow
            maxacc[pl.ds(i * HD + f * L, L)] = neg
        return 0

    lax.fori_loop(0, NPT + 1, init_acc, 0)
    plsc.subcore_barrier()

    pltpu.sync_copy(ctot_h.at[pl.ds(wid * G2, G2)], tot_v)
    total = tot_v[pl.ds(0, L)][0]
    nch = total // G2

    def stage(g):
        gb = (g % 2) * G2
        goff = pl.multiple_of(g * G2, G2)
        a1 = pltpu.async_copy(csx_h.at[wid].at[pl.ds(goff, G2)],
                              csb.at[pl.ds(gb, G2)], sem2)
        a2 = pltpu.async_copy(cdx_h.at[wid].at[pl.ds(goff, G2)],
                              cdb.at[pl.ds(gb, G2)], sem2)
        a1.wait()
        a2.wait()

    def fire(g):
        gb = (g % 2) * G2
        for j in range(G2 // L):
            pltpu.async_copy(sp_h.at[csb.at[pl.ds(gb + j * L, L)]],
                             rows.at[pl.ds(gb + j * L, L)], sem)

    def drain(g):
        gb = (g % 2) * G2
        for j in range(G2 // L):
            pltpu.make_async_copy(sp_h.at[csb.at[pl.ds(gb + j * L, L)]],
                                  rows.at[pl.ds(gb + j * L, L)], sem).wait()

    @pl.when(nch > 0)
    def _():
        stage(0)
        fire(0)

    def chunk(g, _):
        drain(g)

        @pl.when(g + 1 < nch)
        def _():
            stage(g + 1)
            fire(g + 1)

        gb = (g % 2) * G2

        def grp(t, _):
            t0 = t * L
            dvp = cdb[pl.ds(gb + t0, L)]
            for l in range(L):
                v = dvp[l]
                dj = v // 2
                poff = (v & 1) * HD
                for f in range(FG):
                    r = rows[gb + t0 + l, pl.ds(poff + f * L, L)]
                    sumacc[pl.ds(dj * HD + f * L, L)] += r
                    a = maxacc[pl.ds(dj * HD + f * L, L)]
                    maxacc[pl.ds(dj * HD + f * L, L)] = jnp.maximum(a, r)
            return 0

        lax.fori_loop(0, G2 // L, grp, 0)
        return 0

    lax.fori_loop(0, nch, chunk, 0)

    pltpu.sync_copy(sumacc.at[pl.ds(0, NPT * HD)], sum_o.at[pl.ds(lo * HD, NPT * HD)])
    pltpu.sync_copy(maxacc.at[pl.ds(0, NPT * HD)], max_o.at[pl.ds(lo * HD, NPT * HD)])


_ROWS_BLK = 1000
_GRID = N // _ROWS_BLK


def _tc_linear_body(s0, s1, m0, m1, cdeg, w00, w01, w10, w11, b2, sn,
                    h2_ref, o1, o2):
    deg = cdeg[:, :1]
    inv = 1.0 / jnp.maximum(deg, 1.0)
    pos = deg > 0.0
    mean0 = s0[...] * inv
    mean1 = s1[...] * inv
    mx0 = jnp.where(pos, m0[...], 0.0)
    mx1 = jnp.where(pos, m1[...], 0.0)
    h2 = (
        jnp.dot(mean0, w00[...], preferred_element_type=jnp.float32)
        + jnp.dot(mean1, w01[...], preferred_element_type=jnp.float32)
        + jnp.dot(mx0, w10[...], preferred_element_type=jnp.float32)
        + jnp.dot(mx1, w11[...], preferred_element_type=jnp.float32)
        + b2[...]
    ) * sn[...]
    h2_ref[...] = h2

    @pl.when(pl.program_id(0) == 0)
    def _():
        o1[...] = jnp.zeros_like(o1)
        o2[...] = jnp.zeros_like(o2)

    o1[...] += jnp.sum(h2, axis=0, keepdims=True)
    o2[...] += jnp.sum(h2 * h2, axis=0, keepdims=True)


def _tc_norm_body(h2, h, s1, s2, gamma2, beta2, out):
    mu = s1[...] / N
    var = s2[...] / N - mu * mu
    scale = gamma2[...] * lax.rsqrt(var + EPS)
    out[...] = h[...] + jnp.maximum(scale * (h2[...] - mu) + beta2[...], 0.0)


def kernel(h, e, eig, snorm_n, edge_index, W, b, gamma, beta):
    src = edge_index[0].astype(jnp.int32)
    dst = edge_index[1].astype(jnp.int32)

    s0f, c0f, m0f, csx, cdx, ctot = _sc_half(src, dst, h[:, :HD].reshape(N // 2, D))
    s1f, m1f = _sc_pass2(csx, cdx, ctot, h[:, HD:].reshape(N // 2, D))

    s0 = s0f.reshape(NT, HD)[:N]
    s1 = s1f.reshape(NT, HD)[:N]
    m0 = m0f.reshape(NT, HD)[:N]
    m1 = m1f.reshape(NT, HD)[:N]
    cdeg = c0f.reshape(NT, L)[:N]

    rb = lambda i: (i, 0)
    fb = lambda i: (0, 0)
    h2, t1, t2 = pl.pallas_call(
        _tc_linear_body,
        grid=(_GRID,),
        in_specs=[
            pl.BlockSpec((_ROWS_BLK, HD), rb),
            pl.BlockSpec((_ROWS_BLK, HD), rb),
            pl.BlockSpec((_ROWS_BLK, HD), rb),
            pl.BlockSpec((_ROWS_BLK, HD), rb),
            pl.BlockSpec((_ROWS_BLK, L), rb),
            pl.BlockSpec((HD, D), fb),
            pl.BlockSpec((HD, D), fb),
            pl.BlockSpec((HD, D), fb),
            pl.BlockSpec((HD, D), fb),
            pl.BlockSpec((1, D), fb),
            pl.BlockSpec((_ROWS_BLK, 1), rb),
        ],
        out_specs=[
            pl.BlockSpec((_ROWS_BLK, D), rb),
            pl.BlockSpec((1, D), fb),
            pl.BlockSpec((1, D), fb),
        ],
        out_shape=[
            jax.ShapeDtypeStruct((N, D), jnp.float32),
            jax.ShapeDtypeStruct((1, D), jnp.float32),
            jax.ShapeDtypeStruct((1, D), jnp.float32),
        ],
    )(s0, s1, m0, m1, cdeg, W[:HD], W[HD:D], W[D:D + HD], W[D + HD:],
      b.reshape(1, D), snorm_n)

    out = pl.pallas_call(
        _tc_norm_body,
        grid=(_GRID,),
        in_specs=[
            pl.BlockSpec((_ROWS_BLK, D), rb),
            pl.BlockSpec((_ROWS_BLK, D), rb),
            pl.BlockSpec((1, D), fb),
            pl.BlockSpec((1, D), fb),
            pl.BlockSpec((1, D), fb),
            pl.BlockSpec((1, D), fb),
        ],
        out_specs=pl.BlockSpec((_ROWS_BLK, D), rb),
        out_shape=jax.ShapeDtypeStruct((N, D), jnp.float32),
    )(h2, h, t1, t2, gamma.reshape(1, D), beta.reshape(1, D))

    return out


# speedup vs baseline: 4.8262x; 1.0019x over previous
import functools

import jax
import jax.numpy as jnp
from jax import lax
from jax.experimental import pallas as pl
from jax.experimental.pallas import tpu as pltpu
from jax.experimental.pallas import tpu_sc as plsc

N = 10000
E = 320000
D = 128
HD = D // 2
EPS = 1e-5

NC = 2
NS = 16
NW = NC * NS
L = 16

NPT = 320
NT = NPT * NW
BLK = 4000
NB = E // BLK
G = 64
CSZ = BLK + 2 * G + 2 * L
G2 = 2 * G
DUMMY = 1 << 29
FG = HD // L
EP = E + NB * 2 * G

_mesh = plsc.VectorSubcoreMesh(
    core_axis_name="c", subcore_axis_name="s", num_cores=NC, num_subcores=NS
)


@functools.partial(
    pl.kernel,
    compiler_params=pltpu.CompilerParams(needs_layout_passes=False),
    out_type=[
        jax.ShapeDtypeStruct((NT * HD,), jnp.float32),
        jax.ShapeDtypeStruct((NT * L,), jnp.float32),
        jax.ShapeDtypeStruct((NT * HD,), jnp.float32),
        jax.ShapeDtypeStruct((NW, EP), jnp.int32),
        jax.ShapeDtypeStruct((NW, EP), jnp.int32),
        jax.ShapeDtypeStruct((NW * G2,), jnp.int32),
    ],
    mesh=_mesh,
    scratch_types=[
        pltpu.VMEM((BLK,), jnp.int32),
        pltpu.VMEM((BLK,), jnp.int32),
        pltpu.VMEM((CSZ,), jnp.int32),
        pltpu.VMEM((CSZ,), jnp.int32),
        pltpu.VMEM((2 * G, D), jnp.float32),
        pltpu.VMEM(((NPT + 1) * HD,), jnp.float32),
        pltpu.VMEM(((NPT + 1) * HD,), jnp.float32),
        pltpu.VMEM(((NPT + 1) * L,), jnp.float32),
        pltpu.VMEM_SHARED((N // 2, D), jnp.float32),
        pltpu.SemaphoreType.DMA,
        pltpu.SemaphoreType.DMA,
    ],
)
def _sc_half(src_h, dst_h, h_h,
             sum_o, cnt_o, max_o, csx_o, cdx_o, ctot_o,
             srcb, dstb, cs, cd, rows, sumacc, maxacc, cntacc, sp_h, sem, sem2):
    c = lax.axis_index("c")
    s = lax.axis_index("s")
    wid = c * NS + s
    lo = wid * NPT

    neg = jnp.full((L,), -jnp.inf, jnp.float32)
    zrow = jnp.zeros((L,), jnp.float32)

    @pl.when(s == 0)
    def _():
        pltpu.sync_copy(h_h, sp_h)

    def init_acc(i, _):
        for f in range(FG):
            sumacc[pl.ds(i * HD + f * L, L)] = zrow
            maxacc[pl.ds(i * HD + f * L, L)] = neg
        cntacc[pl.ds(i * L, L)] = zrow
        return 0

    lax.fori_loop(0, NPT + 1, init_acc, 0)
    plsc.subcore_barrier()

    iota = lax.broadcasted_iota(jnp.int32, (L,), 0)
    padsrc16 = wid * 128 + iota * 4
    dummy16 = jnp.full((L,), 2 * NPT, jnp.int32)
    one16 = jnp.full((L,), 1.0, jnp.float32)

    def block_body(bi, ofs):
        e0 = bi * BLK
        d1 = pltpu.async_copy(src_h.at[pl.ds(e0, BLK)], srcb, sem2)
        d2 = pltpu.async_copy(dst_h.at[pl.ds(e0, BLK)], dstb, sem2)
        d1.wait()
        d2.wait()

        def comp(i, cnt):
            d = dstb[pl.ds(i * L, L)]
            sv = srcb[pl.ds(i * L, L)]
            m = (d >= lo) & (d < lo + NPT)
            csum = plsc.cumsum(m.astype(jnp.int32))
            pos = jnp.maximum(cnt + csum - 1, 0)
            plsc.store_scatter(cs, [pos], sv // 2, mask=m)
            plsc.store_scatter(cd, [pos], (d - lo) * 2 + (sv & 1), mask=m)
            return cnt + csum[L - 1]

        cnt = lax.fori_loop(0, BLK // L, comp, 0, unroll=4)

        base = (cnt // L) * L
        for k in range(2 * G // L + 1):
            lanes = base + k * L + iota
            m = lanes >= cnt
            plsc.store_scatter(cs, [lanes], padsrc16, mask=m)
            plsc.store_scatter(cd, [lanes], dummy16, mask=m)

        cnt128 = ((cnt + G2 - 1) // G2) * G2
        ng = cnt128 // G

        def fire(g):
            gb = (g % 2) * G
            for j in range(G // L):
                pltpu.async_copy(
                    sp_h.at[cs.at[pl.ds(g * G + j * L, L)]],
                    rows.at[pl.ds(gb + j * L, L)],
                    sem,
                )

        def drain(g):
            gb = (g % 2) * G
            for j in range(G // L):
                pltpu.make_async_copy(
                    sp_h.at[cs.at[pl.ds(g * G + j * L, L)]],
                    rows.at[pl.ds(gb + j * L, L)],
                    sem,
                ).wait()

        @pl.when(ng > 0)
        def _():
            fire(0)

        def chunk(g, _):
            drain(g)

            @pl.when(g + 1 < ng)
            def _():
                fire(g + 1)

            gb = (g % 2) * G

            def grp(t, _):
                t0 = t * L
                dvp = cd[pl.ds(g * G + t0, L)]
                for l in range(L):
                    v = dvp[l]
                    dj = v // 2
                    poff = (v & 1) * HD
                    cntacc[pl.ds(dj * L, L)] += one16
                    for f in range(FG):
                        r = rows[gb + t0 + l, pl.ds(poff + f * L, L)]
                        sumacc[pl.ds(dj * HD + f * L, L)] += r
                        a = maxacc[pl.ds(dj * HD + f * L, L)]
                        maxacc[pl.ds(dj * HD + f * L, L)] = jnp.maximum(a, r)
                return 0

            lax.fori_loop(0, G // L, grp, 0)
            return 0

        lax.fori_loop(0, ng, chunk, 0)

        def wb(w, _):
            off = pl.multiple_of(ofs + w * G2, G2)
            pltpu.sync_copy(cs.at[pl.ds(w * G2, G2)],
                            csx_o.at[wid].at[pl.ds(off, G2)])
            pltpu.sync_copy(cd.at[pl.ds(w * G2, G2)],
                            cdx_o.at[wid].at[pl.ds(off, G2)])
            return 0

        lax.fori_loop(0, cnt128 // G2, wb, 0)
        return ofs + cnt128

    total = lax.fori_loop(0, NB, block_body, 0)
    tvec = jnp.full((L,), 1, jnp.int32) * total
    for k in range(G2 // L):
        srcb[pl.ds(k * L, L)] = tvec
    pltpu.sync_copy(srcb.at[pl.ds(0, G2)], ctot_o.at[pl.ds(wid * G2, G2)])

    pltpu.sync_copy(sumacc.at[pl.ds(0, NPT * HD)], sum_o.at[pl.ds(lo * HD, NPT * HD)])
    pltpu.sync_copy(cntacc.at[pl.ds(0, NPT * L)], cnt_o.at[pl.ds(lo * L, NPT * L)])
    pltpu.sync_copy(maxacc.at[pl.ds(0, NPT * HD)], max_o.at[pl.ds(lo * HD, NPT * HD)])


@functools.partial(
    pl.kernel,
    compiler_params=pltpu.CompilerParams(needs_layout_passes=False),
    out_type=[
        jax.ShapeDtypeStruct((NT * HD,), jnp.float32),
        jax.ShapeDtypeStruct((NT * HD,), jnp.float32),
    ],
    mesh=_mesh,
    scratch_types=[
        pltpu.VMEM((2 * G2,), jnp.int32),
        pltpu.VMEM((2 * G2,), jnp.int32),
        pltpu.VMEM((2 * G2, D), jnp.float32),
        pltpu.VMEM(((NPT + 1) * HD,), jnp.float32),
        pltpu.VMEM(((NPT + 1) * HD,), jnp.float32),
        pltpu.VMEM((G2,), jnp.int32),
        pltpu.VMEM_SHARED((N // 2, D), jnp.float32),
        pltpu.SemaphoreType.DMA,
        pltpu.SemaphoreType.DMA,
    ],
)
def _sc_pass2(csx_h, cdx_h, ctot_h, h_h,
              sum_o, max_o,
              csb, cdb, rows, sumacc, maxacc, tot_v, sp_h, sem, sem2):
    c = lax.axis_index("c")
    s = lax.axis_index("s")
    wid = c * NS + s
    lo = wid * NPT

    neg = jnp.full((L,), -jnp.inf, jnp.float32)
    zrow = jnp.zeros((L,), jnp.float32)

    @pl.when(s == 0)
    def _():
        pltpu.sync_copy(h_h, sp_h)

    def init_acc(i, _):
        for f in range(FG):
            sumacc[pl.ds(i * HD + f * L, L)] = zrow
            maxacc[pl.ds(i * HD + f * L, L)] = neg
        return 0

    lax.fori_loop(0, NPT + 1, init_acc, 0)
    plsc.subcore_barrier()

    pltpu.sync_copy(ctot_h.at[pl.ds(wid * G2, G2)], tot_v)
    total = tot_v[pl.ds(0, L)][0]
    nch = total // G2

    def stage(g):
        gb = (g % 2) * G2
        goff = pl.multiple_of(g * G2, G2)
        a1 = pltpu.async_copy(csx_h.at[wid].at[pl.ds(goff, G2)],
                              csb.at[pl.ds(gb, G2)], sem2)
        a2 = pltpu.async_copy(cdx_h.at[wid].at[pl.ds(goff, G2)],
                              cdb.at[pl.ds(gb, G2)], sem2)
        a1.wait()
        a2.wait()

    def fire(g):
        gb = (g % 2) * G2
        for j in range(G2 // L):
            pltpu.async_copy(sp_h.at[csb.at[pl.ds(gb + j * L, L)]],
                             rows.at[pl.ds(gb + j * L, L)], sem)

    def drain(g):
        gb = (g % 2) * G2
        for j in range(G2 // L):
            pltpu.make_async_copy(sp_h.at[csb.at[pl.ds(gb + j * L, L)]],
                                  rows.at[pl.ds(gb + j * L, L)], sem).wait()

    @pl.when(nch > 0)
    def _():
        stage(0)
        fire(0)

    def chunk(g, _):
        drain(g)

        @pl.when(g + 1 < nch)
        def _():
            stage(g + 1)
            fire(g + 1)

        gb = (g % 2) * G2

        def grp(t, _):
            t0 = t * L
            dvp = cdb[pl.ds(gb + t0, L)]
            for l in range(L):
                v = dvp[l]
                dj = v // 2
                poff = (v & 1) * HD
                for f in range(FG):
                    r = rows[gb + t0 + l, pl.ds(poff + f * L, L)]
                    sumacc[pl.ds(dj * HD + f * L, L)] += r
                    a = maxacc[pl.ds(dj * HD + f * L, L)]
                    maxacc[pl.ds(dj * HD + f * L, L)] = jnp.maximum(a, r)
            return 0

        lax.fori_loop(0, G2 // L, grp, 0)
        return 0

    lax.fori_loop(0, nch, chunk, 0)

    pltpu.sync_copy(sumacc.at[pl.ds(0, NPT * HD)], sum_o.at[pl.ds(lo * HD, NPT * HD)])
    pltpu.sync_copy(maxacc.at[pl.ds(0, NPT * HD)], max_o.at[pl.ds(lo * HD, NPT * HD)])


_ROWS_BLK = 1000
_GRID = N // _ROWS_BLK


def _tc_linear_body(s0, s1, m0, m1, cdeg, w00, w01, w10, w11, b2, sn,
                    h2_ref, o1, o2):
    deg = cdeg[:, :1]
    inv = 1.0 / jnp.maximum(deg, 1.0)
    pos = deg > 0.0
    mean0 = s0[...] * inv
    mean1 = s1[...] * inv
    mx0 = jnp.where(pos, m0[...], 0.0)
    mx1 = jnp.where(pos, m1[...], 0.0)
    h2 = (
        jnp.dot(mean0, w00[...], preferred_element_type=jnp.float32)
        + jnp.dot(mean1, w01[...], preferred_element_type=jnp.float32)
        + jnp.dot(mx0, w10[...], preferred_element_type=jnp.float32)
        + jnp.dot(mx1, w11[...], preferred_element_type=jnp.float32)
        + b2[...]
    ) * sn[...]
    h2_ref[...] = h2

    @pl.when(pl.program_id(0) == 0)
    def _():
        o1[...] = jnp.zeros_like(o1)
        o2[...] = jnp.zeros_like(o2)

    o1[...] += jnp.sum(h2, axis=0, keepdims=True)
    o2[...] += jnp.sum(h2 * h2, axis=0, keepdims=True)


def _tc_norm_body(h2, h, s1, s2, gamma2, beta2, out):
    mu = s1[...] / N
    var = s2[...] / N - mu * mu
    scale = gamma2[...] * lax.rsqrt(var + EPS)
    out[...] = h[...] + jnp.maximum(scale * (h2[...] - mu) + beta2[...], 0.0)


def kernel(h, e, eig, snorm_n, edge_index, W, b, gamma, beta):
    src = edge_index[0].astype(jnp.int32)
    dst = edge_index[1].astype(jnp.int32)

    s0f, c0f, m0f, csx, cdx, ctot = _sc_half(src, dst, h[:, :HD].reshape(N // 2, D))
    s1f, m1f = _sc_pass2(csx, cdx, ctot, h[:, HD:].reshape(N // 2, D))

    s0 = s0f.reshape(NT, HD)[:N]
    s1 = s1f.reshape(NT, HD)[:N]
    m0 = m0f.reshape(NT, HD)[:N]
    m1 = m1f.reshape(NT, HD)[:N]
    cdeg = c0f.reshape(NT, L)[:N]

    rb = lambda i: (i, 0)
    fb = lambda i: (0, 0)
    h2, t1, t2 = pl.pallas_call(
        _tc_linear_body,
        grid=(_GRID,),
        in_specs=[
            pl.BlockSpec((_ROWS_BLK, HD), rb),
            pl.BlockSpec((_ROWS_BLK, HD), rb),
            pl.BlockSpec((_ROWS_BLK, HD), rb),
            pl.BlockSpec((_ROWS_BLK, HD), rb),
            pl.BlockSpec((_ROWS_BLK, L), rb),
            pl.BlockSpec((HD, D), fb),
            pl.BlockSpec((HD, D), fb),
            pl.BlockSpec((HD, D), fb),
            pl.BlockSpec((HD, D), fb),
            pl.BlockSpec((1, D), fb),
            pl.BlockSpec((_ROWS_BLK, 1), rb),
        ],
        out_specs=[
            pl.BlockSpec((_ROWS_BLK, D), rb),
            pl.BlockSpec((1, D), fb),
            pl.BlockSpec((1, D), fb),
        ],
        out_shape=[
            jax.ShapeDtypeStruct((N, D), jnp.float32),
            jax.ShapeDtypeStruct((1, D), jnp.float32),
            jax.ShapeDtypeStruct((1, D), jnp.float32),
        ],
    )(s0, s1, m0, m1, cdeg, W[:HD], W[HD:D], W[D:D + HD], W[D + HD:],
      b.reshape(1, D), snorm_n)

    out = pl.pallas_call(
        _tc_norm_body,
        grid=(_GRID,),
        in_specs=[
            pl.BlockSpec((_ROWS_BLK, D), rb),
            pl.BlockSpec((_ROWS_BLK, D), rb),
            pl.BlockSpec((1, D), fb),
            pl.BlockSpec((1, D), fb),
            pl.BlockSpec((1, D), fb),
            pl.BlockSpec((1, D), fb),
        ],
        out_specs=pl.BlockSpec((_ROWS_BLK, D), rb),
        out_shape=jax.ShapeDtypeStruct((N, D), jnp.float32),
    )(h2, h, t1, t2, gamma.reshape(1, D), beta.reshape(1, D))

    return out
